# Initial kernel scaffold; baseline (speedup 1.0000x reference)
import jax
import jax.numpy as jnp
from jax.experimental import pallas as pl


def kernel(v1, v2, weight, event1, event2, batter_idx, pitcher_idx):
    raise NotImplementedError("write your pallas kernel here")


# speedup vs baseline: 155.0112x; 155.0112x over previous
import functools
import math

import jax
import jax.numpy as jnp
from jax import lax
from jax.experimental import pallas as pl
from jax.experimental.pallas import tpu as pltpu
from jax.experimental.pallas import tpu_sc as plsc

_N_EV = 1048576
_LANES = 16
_NC = 2
_NS = 16
_NW = _NC * _NS
_EV_PER_TILE = _N_EV // _NW
_CHUNK = 8192
_NCHUNKS = _EV_PER_TILE // _CHUNK

_LOG_LO = math.log(1e-6)
_LOG_HI = math.log(0.999999)


def _loss_terms(x):
    u = jnp.exp(-jnp.abs(x))
    z = u / (u + 2.0)
    z2 = z * z
    poly = 1.0 + z2 * (0.33333334 + z2 * (0.2 + z2 * (0.14285715 + z2 * 0.11111111)))
    sp = jnp.maximum(x, 0.0) + (2.0 * z) * poly
    lp = jnp.minimum(jnp.maximum(x - sp, _LOG_LO), _LOG_HI)
    lq = jnp.minimum(jnp.maximum(-sp, _LOG_LO), _LOG_HI)
    return lp, lq


def _sc_body(v1h, v2h, wh, e1h, e2h, bih, pih, outh,
             bidxv, pidxv, e1v, e2v, bv, pv, wv, accv, s1, s2):
    cid = lax.axis_index("c")
    sid = lax.axis_index("s")
    wid = cid * _NS + sid
    ev0 = wid * _EV_PER_TILE

    pltpu.sync_copy(wh, wv)
    w = wv[...]
    omw = 1.0 - w

    def chunk_body(ci, acc):
        base = ev0 + ci * _CHUNK
        pltpu.sync_copy(bih.at[pl.ds(base, _CHUNK)], bidxv)
        pltpu.sync_copy(pih.at[pl.ds(base, _CHUNK)], pidxv)
        pltpu.sync_copy(e1h.at[pl.ds(base, _CHUNK)], e1v)
        pltpu.sync_copy(e2h.at[pl.ds(base, _CHUNK)], e2v)
        g1 = pltpu.async_copy(v1h.at[bidxv], bv, s1)
        g2 = pltpu.async_copy(v2h.at[pidxv], pv, s2)
        g1.wait()
        g2.wait()

        def row_body(r, acc_r):
            a = acc_r
            for c8 in range(8):
                sl = pl.ds(r * 128 + c8 * _LANES, _LANES)
                x = w * bv[sl] + omw * pv[sl]
                lp, lq = _loss_terms(x)
                a = a + e1v[sl] * lp + e2v[sl] * lq
            return a

        return lax.fori_loop(0, _CHUNK // 128, row_body, acc)

    acc = lax.fori_loop(0, _NCHUNKS, chunk_body, jnp.zeros((_LANES,), jnp.float32))
    accv[...] = acc
    pltpu.sync_copy(accv, outh.at[wid])


_sc_call = functools.partial(
    pl.kernel,
    out_type=jax.ShapeDtypeStruct((_NW, _LANES), jnp.float32),
    mesh=plsc.VectorSubcoreMesh(core_axis_name="c", subcore_axis_name="s"),
    scratch_types=[
        pltpu.VMEM((_CHUNK,), jnp.int32),
        pltpu.VMEM((_CHUNK,), jnp.int32),
        pltpu.VMEM((_CHUNK,), jnp.float32),
        pltpu.VMEM((_CHUNK,), jnp.float32),
        pltpu.VMEM((_CHUNK,), jnp.float32),
        pltpu.VMEM((_CHUNK,), jnp.float32),
        pltpu.VMEM((_LANES,), jnp.float32),
        pltpu.VMEM((_LANES,), jnp.float32),
        pltpu.SemaphoreType.DMA,
        pltpu.SemaphoreType.DMA,
    ],
)(_sc_body)


def kernel(v1, v2, weight, event1, event2, batter_idx, pitcher_idx):
    w16 = jnp.broadcast_to(weight.astype(jnp.float32), (_LANES,))
    parts = _sc_call(v1, v2, w16, event1, event2, batter_idx, pitcher_idx)
    return jnp.sum(parts)


# speedup vs baseline: 191.3683x; 1.2345x over previous
import functools
import math

import jax
import jax.numpy as jnp
from jax import lax
from jax.experimental import pallas as pl
from jax.experimental.pallas import tpu as pltpu
from jax.experimental.pallas import tpu_sc as plsc

_N_EV = 1048576
_N_TAB = 100000
_LANES = 16
_NC = 2
_NS = 16
_NW = _NC * _NS
_EV_PER_TILE = _N_EV // _NW
_CHUNK = 2048
_NCHUNKS = _EV_PER_TILE // _CHUNK
_PAIRS = _CHUNK // (2 * _LANES)

_LOG_LO = math.log(1e-6)
_LOG_HI = math.log(0.999999)


def _loss_terms(x):
    u = jnp.exp(-jnp.abs(x))
    z = u / (u + 2.0)
    z2 = z * z
    poly = 1.0 + z2 * (0.33333334 + z2 * (0.2 + z2 * (0.14285715 + z2 * 0.11111111)))
    sp = jnp.maximum(x, 0.0) + (2.0 * z) * poly
    lp = jnp.minimum(jnp.maximum(x - sp, _LOG_LO), _LOG_HI)
    lq = jnp.minimum(jnp.maximum(-sp, _LOG_LO), _LOG_HI)
    return lp, lq


def _sc_body(v1h, v2h, wh, e1h, e2h, bih, pih, outh,
             tabv, ball, idxv, e1v, e2v, wv, accv):
    cid = lax.axis_index("c")
    sid = lax.axis_index("s")
    wid = cid * _NS + sid
    ev0 = wid * _EV_PER_TILE

    pltpu.sync_copy(wh, wv)
    w = wv[...]
    omw = 1.0 - w

    pltpu.sync_copy(v1h, tabv)

    def p1_chunk(ci, carry):
        base = ev0 + ci * _CHUNK
        pltpu.sync_copy(bih.at[pl.ds(base, _CHUNK)], idxv)

        def p1_vreg(r, c2):
            off = r * (2 * _LANES)
            lo = plsc.load_gather(tabv, [idxv[pl.ds(off, _LANES)]])
            hi = plsc.load_gather(tabv, [idxv[pl.ds(off + _LANES, _LANES)]])
            ball[pl.ds(ci * _CHUNK + off, 2 * _LANES)] = plsc.pack(
                lo, hi, format=plsc.PackFormat.INTERLEAVED)
            return c2

        return lax.fori_loop(0, _PAIRS, p1_vreg, carry)

    lax.fori_loop(0, _NCHUNKS, p1_chunk, 0)

    pltpu.sync_copy(v2h, tabv)

    def p2_chunk(ci, acc):
        base = ev0 + ci * _CHUNK
        pltpu.sync_copy(pih.at[pl.ds(base, _CHUNK)], idxv)
        pltpu.sync_copy(e1h.at[pl.ds(base, _CHUNK)], e1v)
        pltpu.sync_copy(e2h.at[pl.ds(base, _CHUNK)], e2v)

        def p2_vreg(r, acc_r):
            off = r * (2 * _LANES)
            b_lo, b_hi = plsc.unpack(
                ball[pl.ds(ci * _CHUNK + off, 2 * _LANES)],
                format=plsc.PackFormat.INTERLEAVED)
            a = acc_r
            for half, bb in ((0, b_lo), (_LANES, b_hi)):
                sl = pl.ds(off + half, _LANES)
                p16 = plsc.load_gather(tabv, [idxv[sl]])
                x = w * bb.astype(jnp.float32) + omw * p16
                lp, lq = _loss_terms(x)
                a = a + e1v[sl] * lp + e2v[sl] * lq
            return a

        return lax.fori_loop(0, _PAIRS, p2_vreg, acc)

    acc = lax.fori_loop(0, _NCHUNKS, p2_chunk, jnp.zeros((_LANES,), jnp.float32))
    accv[...] = acc
    pltpu.sync_copy(accv, outh.at[wid])


_sc_call = functools.partial(
    pl.kernel,
    out_type=jax.ShapeDtypeStruct((_NW, _LANES), jnp.float32),
    mesh=plsc.VectorSubcoreMesh(core_axis_name="c", subcore_axis_name="s"),
    compiler_params=pltpu.CompilerParams(needs_layout_passes=False),
    scratch_types=[
        pltpu.VMEM((_N_TAB,), jnp.float32),
        pltpu.VMEM((_EV_PER_TILE,), jnp.bfloat16),
        pltpu.VMEM((_CHUNK,), jnp.int32),
        pltpu.VMEM((_CHUNK,), jnp.float32),
        pltpu.VMEM((_CHUNK,), jnp.float32),
        pltpu.VMEM((_LANES,), jnp.float32),
        pltpu.VMEM((_LANES,), jnp.float32),
    ],
)(_sc_body)


def kernel(v1, v2, weight, event1, event2, batter_idx, pitcher_idx):
    w16 = jnp.broadcast_to(weight.astype(jnp.float32), (_LANES,))
    parts = _sc_call(v1, v2, w16, event1, event2, batter_idx, pitcher_idx)
    return jnp.sum(parts)


# speedup vs baseline: 274.3264x; 1.4335x over previous
import functools
import math

import jax
import jax.numpy as jnp
from jax import lax
from jax.experimental import pallas as pl
from jax.experimental.pallas import tpu as pltpu
from jax.experimental.pallas import tpu_sc as plsc

_N_EV = 1048576
_N_TAB = 100000
_LANES = 16
_NC = 2
_NS = 16
_NW = _NC * _NS
_EV_PER_TILE = _N_EV // _NW
_CHUNK = 2048
_NCHUNKS = _EV_PER_TILE // _CHUNK
_PAIRS = _CHUNK // (2 * _LANES)

_XCLIP = -math.log(1e-6)


def _softplus(x):
    u = jnp.exp(-jnp.abs(x))
    z = u / (u + 2.0)
    z2 = z * z
    poly = 1.0 + z2 * (0.33333334 + z2 * (0.2 + z2 * (0.14285715 + z2 * 0.11111111)))
    return jnp.maximum(x, 0.0) + (2.0 * z) * poly


def _sc_body(v1h, v2h, wh, e1h, e2h, bih, pih, outh,
             tabv, ball, idx0, idx1, e10, e11, e20, e21, wv, accv,
             s0, s1, st):
    cid = lax.axis_index("c")
    sid = lax.axis_index("s")
    wid = cid * _NS + sid
    ev0 = wid * _EV_PER_TILE

    idxs = (idx0, idx1)
    e1s = (e10, e11)
    e2s = (e20, e21)
    sems = (s0, s1)

    pltpu.sync_copy(wh, wv)
    w = wv[...]
    omw = 1.0 - w

    tab_cp = pltpu.async_copy(v1h, tabv, st)

    def start_idx(ci, src):
        k = ci % 2
        base = ev0 + ci * _CHUNK
        return pltpu.async_copy(src.at[pl.ds(base, _CHUNK)], idxs[k], sems[k])

    pend = start_idx(0, bih)
    tab_cp.wait()

    for ci in range(_NCHUNKS):
        idxv = idxs[ci % 2]
        cur = pend
        if ci + 1 < _NCHUNKS:
            pend = start_idx(ci + 1, bih)
        cur.wait()

        def p1_vreg(r, c2, _idxv=idxv, _ci=ci):
            off = r * (2 * _LANES)
            lo = plsc.load_gather(tabv, [_idxv[pl.ds(off, _LANES)]])
            hi = plsc.load_gather(tabv, [_idxv[pl.ds(off + _LANES, _LANES)]])
            ball[pl.ds(_ci * _CHUNK + off, 2 * _LANES)] = plsc.pack(
                lo, hi, format=plsc.PackFormat.INTERLEAVED)
            return c2

        lax.fori_loop(0, _PAIRS, p1_vreg, 0)

    tab_cp = pltpu.async_copy(v2h, tabv, st)

    def start_ev(ci):
        k = ci % 2
        base = ev0 + ci * _CHUNK
        c_i = pltpu.async_copy(pih.at[pl.ds(base, _CHUNK)], idxs[k], sems[k])
        c_1 = pltpu.async_copy(e1h.at[pl.ds(base, _CHUNK)], e1s[k], sems[k])
        c_2 = pltpu.async_copy(e2h.at[pl.ds(base, _CHUNK)], e2s[k], sems[k])
        return (c_i, c_1, c_2)

    pend = start_ev(0)
    tab_cp.wait()

    acc_lo = jnp.zeros((_LANES,), jnp.float32)
    acc_hi = jnp.zeros((_LANES,), jnp.float32)
    for ci in range(_NCHUNKS):
        k = ci % 2
        idxv, e1v, e2v = idxs[k], e1s[k], e2s[k]
        cur = pend
        if ci + 1 < _NCHUNKS:
            pend = start_ev(ci + 1)
        for c in cur:
            c.wait()

        def p2_vreg(r, accs, _refs=(idxv, e1v, e2v), _ci=ci):
            _idxv, _e1v, _e2v = _refs
            off = r * (2 * _LANES)
            b_lo, b_hi = plsc.unpack(
                ball[pl.ds(_ci * _CHUNK + off, 2 * _LANES)],
                format=plsc.PackFormat.INTERLEAVED)
            out = []
            for half, bb, a in ((0, b_lo, accs[0]), (_LANES, b_hi, accs[1])):
                sl = pl.ds(off + half, _LANES)
                p16 = plsc.load_gather(tabv, [_idxv[sl]])
                x = w * bb.astype(jnp.float32) + omw * p16
                x = jnp.minimum(jnp.maximum(x, -_XCLIP), _XCLIP)
                sp = _softplus(x)
                e1 = _e1v[sl]
                out.append(a + (e1 * x - (e1 + _e2v[sl]) * sp))
            return tuple(out)

        acc_lo, acc_hi = lax.fori_loop(0, _PAIRS, p2_vreg, (acc_lo, acc_hi))

    accv[...] = acc_lo + acc_hi
    pltpu.sync_copy(accv, outh.at[wid])


_sc_call = functools.partial(
    pl.kernel,
    out_type=jax.ShapeDtypeStruct((_NW, _LANES), jnp.float32),
    mesh=plsc.VectorSubcoreMesh(core_axis_name="c", subcore_axis_name="s"),
    compiler_params=pltpu.CompilerParams(needs_layout_passes=False),
    scratch_types=[
        pltpu.VMEM((_N_TAB,), jnp.float32),
        pltpu.VMEM((_EV_PER_TILE,), jnp.bfloat16),
        pltpu.VMEM((_CHUNK,), jnp.int32),
        pltpu.VMEM((_CHUNK,), jnp.int32),
        pltpu.VMEM((_CHUNK,), jnp.float32),
        pltpu.VMEM((_CHUNK,), jnp.float32),
        pltpu.VMEM((_CHUNK,), jnp.float32),
        pltpu.VMEM((_CHUNK,), jnp.float32),
        pltpu.VMEM((_LANES,), jnp.float32),
        pltpu.VMEM((_LANES,), jnp.float32),
        pltpu.SemaphoreType.DMA,
        pltpu.SemaphoreType.DMA,
        pltpu.SemaphoreType.DMA,
    ],
)(_sc_body)


def kernel(v1, v2, weight, event1, event2, batter_idx, pitcher_idx):
    w16 = jnp.broadcast_to(weight.astype(jnp.float32), (_LANES,))
    parts = _sc_call(v1, v2, w16, event1, event2, batter_idx, pitcher_idx)
    return jnp.sum(parts)


# speedup vs baseline: 280.2228x; 1.0215x over previous
import functools
import math

import jax
import jax.numpy as jnp
from jax import lax
from jax.experimental import pallas as pl
from jax.experimental.pallas import tpu as pltpu
from jax.experimental.pallas import tpu_sc as plsc

_N_EV = 1048576
_N_TAB = 100000
_LANES = 16
_NC = 2
_NS = 16
_NW = _NC * _NS
_EV_PER_TILE = _N_EV // _NW
_CHUNK = 2048
_NCHUNKS = _EV_PER_TILE // _CHUNK
_PAIRS = _CHUNK // (2 * _LANES)

_XCLIP = -math.log(1e-6)


_C7 = (0.010009289719164371, -0.05243753641843796, 0.1308334320783615,
       -0.2231658697128296, 0.3272257149219513, -0.4992850422859192,
       0.999967098236084, 2.554673130816809e-07)


def _softplus(x):
    u = jnp.exp(-jnp.abs(x))
    acc = _C7[0] * u + _C7[1]
    for c in _C7[2:]:
        acc = acc * u + c
    return jnp.maximum(x, 0.0) + acc


def _sc_body(v1h, v2h, wh, e1h, e2h, bih, pih, outh,
             tabv, ball, idx0, idx1, e10, e11, e20, e21, wv, accv,
             s0, s1, st):
    cid = lax.axis_index("c")
    sid = lax.axis_index("s")
    wid = cid * _NS + sid
    ev0 = wid * _EV_PER_TILE

    idxs = (idx0, idx1)
    e1s = (e10, e11)
    e2s = (e20, e21)
    sems = (s0, s1)

    pltpu.sync_copy(wh, wv)
    w = wv[...]
    omw = 1.0 - w

    tab_cp = pltpu.async_copy(v1h, tabv, st)

    def start_idx(ci, src):
        k = ci % 2
        base = ev0 + ci * _CHUNK
        return pltpu.async_copy(src.at[pl.ds(base, _CHUNK)], idxs[k], sems[k])

    pend = start_idx(0, bih)
    tab_cp.wait()

    for ci in range(_NCHUNKS):
        idxv = idxs[ci % 2]
        cur = pend
        if ci + 1 < _NCHUNKS:
            pend = start_idx(ci + 1, bih)
        cur.wait()

        def p1_vreg(r, c2, _idxv=idxv, _ci=ci):
            off = r * (2 * _LANES)
            lo = plsc.load_gather(tabv, [_idxv[pl.ds(off, _LANES)]])
            hi = plsc.load_gather(tabv, [_idxv[pl.ds(off + _LANES, _LANES)]])
            ball[pl.ds(_ci * _CHUNK + off, 2 * _LANES)] = plsc.pack(
                lo, hi, format=plsc.PackFormat.INTERLEAVED)
            return c2

        lax.fori_loop(0, _PAIRS, p1_vreg, 0)

    tab_cp = pltpu.async_copy(v2h, tabv, st)

    def start_ev(ci):
        k = ci % 2
        base = ev0 + ci * _CHUNK
        c_i = pltpu.async_copy(pih.at[pl.ds(base, _CHUNK)], idxs[k], sems[k])
        c_1 = pltpu.async_copy(e1h.at[pl.ds(base, _CHUNK)], e1s[k], sems[k])
        c_2 = pltpu.async_copy(e2h.at[pl.ds(base, _CHUNK)], e2s[k], sems[k])
        return (c_i, c_1, c_2)

    pend = start_ev(0)
    tab_cp.wait()

    acc_lo = jnp.zeros((_LANES,), jnp.float32)
    acc_hi = jnp.zeros((_LANES,), jnp.float32)
    for ci in range(_NCHUNKS):
        k = ci % 2
        idxv, e1v, e2v = idxs[k], e1s[k], e2s[k]
        cur = pend
        if ci + 1 < _NCHUNKS:
            pend = start_ev(ci + 1)
        for c in cur:
            c.wait()

        def p2_vreg(r, accs, _refs=(idxv, e1v, e2v), _ci=ci):
            _idxv, _e1v, _e2v = _refs
            off = r * (2 * _LANES)
            b_lo, b_hi = plsc.unpack(
                ball[pl.ds(_ci * _CHUNK + off, 2 * _LANES)],
                format=plsc.PackFormat.INTERLEAVED)
            out = []
            for half, bb, a in ((0, b_lo, accs[0]), (_LANES, b_hi, accs[1])):
                sl = pl.ds(off + half, _LANES)
                p16 = plsc.load_gather(tabv, [_idxv[sl]])
                x = w * bb.astype(jnp.float32) + omw * p16
                x = jnp.minimum(jnp.maximum(x, -_XCLIP), _XCLIP)
                sp = _softplus(x)
                e1 = _e1v[sl]
                out.append(a + (e1 * x - (e1 + _e2v[sl]) * sp))
            return tuple(out)

        acc_lo, acc_hi = lax.fori_loop(0, _PAIRS, p2_vreg, (acc_lo, acc_hi))

    accv[...] = acc_lo + acc_hi
    pltpu.sync_copy(accv, outh.at[wid])


_sc_call = functools.partial(
    pl.kernel,
    out_type=jax.ShapeDtypeStruct((_NW, _LANES), jnp.float32),
    mesh=plsc.VectorSubcoreMesh(core_axis_name="c", subcore_axis_name="s"),
    compiler_params=pltpu.CompilerParams(needs_layout_passes=False),
    scratch_types=[
        pltpu.VMEM((_N_TAB,), jnp.float32),
        pltpu.VMEM((_EV_PER_TILE,), jnp.bfloat16),
        pltpu.VMEM((_CHUNK,), jnp.int32),
        pltpu.VMEM((_CHUNK,), jnp.int32),
        pltpu.VMEM((_CHUNK,), jnp.float32),
        pltpu.VMEM((_CHUNK,), jnp.float32),
        pltpu.VMEM((_CHUNK,), jnp.float32),
        pltpu.VMEM((_CHUNK,), jnp.float32),
        pltpu.VMEM((_LANES,), jnp.float32),
        pltpu.VMEM((_LANES,), jnp.float32),
        pltpu.SemaphoreType.DMA,
        pltpu.SemaphoreType.DMA,
        pltpu.SemaphoreType.DMA,
    ],
)(_sc_body)


def kernel(v1, v2, weight, event1, event2, batter_idx, pitcher_idx):
    w16 = jnp.broadcast_to(weight.astype(jnp.float32), (_LANES,))
    parts = _sc_call(v1, v2, w16, event1, event2, batter_idx, pitcher_idx)
    return jnp.sum(parts)
